# baseline (speedup 1.0000x reference)
import functools

import jax
import jax.numpy as jnp
from jax import lax
from jax.experimental import pallas as pl
from jax.experimental.pallas import tpu as pltpu
from jax.experimental.pallas import tpu_sc as plsc

N = 10000
E = 320000
D = 128

NC = 2
NS = 16
NW = NC * NS
D2 = D // 2
ACHUNK = 125
NCHUNK = E // NW // ACHUNK
NCHA = E // NS // ACHUNK
NBUF = 4

DSEG = 624
DTAIL = N - NS * DSEG

@functools.cache
def _mesh():
    return plsc.VectorSubcoreMesh(
        core_axis_name="c", subcore_axis_name="s",
        num_cores=NC, num_subcores=NS)


def _zeros16():
    return jnp.zeros((16,), jnp.float32)


def _sc_deg_body(src_hbm, dst_hbm, od_hbm, id_hbm,
                 idxv, idxv2, onesv, zbuf, od_sh, id_sh, dsem, dsem2):
    c = lax.axis_index("c")
    s = lax.axis_index("s")
    w = c * NS + s

    @pl.loop(0, 40)
    def _zero_zbuf(i):
        zbuf[pl.ds(i * 16, 16)] = _zeros16()

    for j in range(128 // 16):
        onesv[pl.ds(j * 16, 16)] = jnp.ones((16,), jnp.float32)

    pltpu.sync_copy(zbuf.at[pl.ds(0, DSEG)], od_sh.at[pl.ds(s * DSEG, DSEG)])
    pltpu.sync_copy(zbuf.at[pl.ds(0, DSEG)], id_sh.at[pl.ds(s * DSEG, DSEG)])

    @pl.when(s == NS - 1)
    def _zero_tail():
        pltpu.sync_copy(zbuf.at[pl.ds(0, DTAIL)],
                        od_sh.at[pl.ds(NS * DSEG, DTAIL)])
        pltpu.sync_copy(zbuf.at[pl.ds(0, DTAIL)],
                        id_sh.at[pl.ds(NS * DSEG, DTAIL)])

    plsc.subcore_barrier()

    pltpu.sync_copy(src_hbm.at[s, pl.ds(c * NCHUNK, NCHUNK)], idxv)
    pltpu.sync_copy(dst_hbm.at[s, pl.ds(c * NCHUNK, NCHUNK)], idxv2)

    ones = onesv.at[pl.ds(0, ACHUNK)]

    @pl.loop(0, NCHUNK, step=8)
    def _scatter_deg(j):
        for b in range(8):
            pltpu.async_copy(ones, od_sh.at[idxv.at[j + b]], dsem, add=True)
            pltpu.async_copy(ones, id_sh.at[idxv2.at[j + b]], dsem2, add=True)
        for b in range(8):
            pltpu.make_async_copy(ones, od_sh.at[idxv.at[j + b]], dsem).wait()
            pltpu.make_async_copy(ones, id_sh.at[idxv2.at[j + b]],
                                  dsem2).wait()

    plsc.subcore_barrier()

    pltpu.sync_copy(od_sh.at[pl.ds(s * DSEG, DSEG)], zbuf.at[pl.ds(0, DSEG)])
    pltpu.sync_copy(zbuf.at[pl.ds(0, DSEG)],
                    od_hbm.at[pl.ds(c * N + s * DSEG, DSEG)])
    pltpu.sync_copy(id_sh.at[pl.ds(s * DSEG, DSEG)], zbuf.at[pl.ds(0, DSEG)])
    pltpu.sync_copy(zbuf.at[pl.ds(0, DSEG)],
                    id_hbm.at[pl.ds(c * N + s * DSEG, DSEG)])

    @pl.when(s == NS - 1)
    def _write_tail():
        pltpu.sync_copy(od_sh.at[pl.ds(NS * DSEG, DTAIL)],
                        zbuf.at[pl.ds(0, DTAIL)])
        pltpu.sync_copy(zbuf.at[pl.ds(0, DTAIL)],
                        od_hbm.at[pl.ds(c * N + NS * DSEG, DTAIL)])
        pltpu.sync_copy(id_sh.at[pl.ds(NS * DSEG, DTAIL)],
                        zbuf.at[pl.ds(0, DTAIL)])
        pltpu.sync_copy(zbuf.at[pl.ds(0, DTAIL)],
                        id_hbm.at[pl.ds(c * N + NS * DSEG, DTAIL)])


@functools.cache
def _sc_deg():
    return pl.kernel(
        _sc_deg_body,
        out_type=[
            jax.ShapeDtypeStruct((NC * N,), jnp.float32),
            jax.ShapeDtypeStruct((NC * N,), jnp.float32),
        ],
        mesh=_mesh(),
        scratch_types=[
            pltpu.VMEM((NCHUNK, ACHUNK), jnp.int32),
            pltpu.VMEM((NCHUNK, ACHUNK), jnp.int32),
            pltpu.VMEM((128,), jnp.float32),
            pltpu.VMEM((640,), jnp.float32),
            pltpu.VMEM_SHARED((N,), jnp.float32),
            pltpu.VMEM_SHARED((N,), jnp.float32),
            pltpu.SemaphoreType.DMA,
            pltpu.SemaphoreType.DMA,
        ],
    )


def _sc_agg_body(h_hbm, src2_hbm, dst_hbm, out_hbm,
                 srcv, dstv, buf0, buf1, buf2, buf3,
                 stg, agg_sh, gsem0, gsem1, gsem2, gsem3,
                 ssem0, ssem1, ssem2, ssem3):
    c = lax.axis_index("c")
    s = lax.axis_index("s")
    bufs = (buf0, buf1, buf2, buf3)
    gsems = (gsem0, gsem1, gsem2, gsem3)
    ssems = (ssem0, ssem1, ssem2, ssem3)

    @pl.loop(0, DSEG // 6)
    def _zero_stg(r):
        for j in range(D2 // 16):
            stg[r, pl.ds(j * 16, 16)] = _zeros16()

    @pl.loop(0, 6)
    def _zero_agg(k):
        pltpu.sync_copy(stg,
                        agg_sh.at[pl.ds(s * DSEG + k * (DSEG // 6), DSEG // 6)])

    @pl.when(s == NS - 1)
    def _zero_tail():
        pltpu.sync_copy(stg.at[pl.ds(0, DTAIL)],
                        agg_sh.at[pl.ds(NS * DSEG, DTAIL)])

    plsc.subcore_barrier()

    pltpu.sync_copy(src2_hbm.at[c, s], srcv)
    pltpu.sync_copy(dst_hbm.at[s], dstv)
    hc = h_hbm

    def start_gather(j, b):
        pltpu.async_copy(hc.at[srcv.at[j]], bufs[b], gsems[b])

    def wait_gather(j, b):
        pltpu.make_async_copy(hc.at[srcv.at[j]], bufs[b], gsems[b]).wait()

    def start_scatter(j, b):
        pltpu.async_copy(bufs[b], agg_sh.at[dstv.at[j]], ssems[b], add=True)

    def wait_scatter(j, b):
        pltpu.make_async_copy(bufs[b], agg_sh.at[dstv.at[j]],
                              ssems[b]).wait()

    for b in range(NBUF):
        start_gather(b, b)

    @pl.loop(0, NCHA - NBUF, step=NBUF)
    def _edge_loop(j):
        for b in range(NBUF):
            wait_gather(j + b, b)
            start_scatter(j + b, b)
        for b in range(NBUF):
            wait_scatter(j + b, b)
            start_gather(j + NBUF + b, b)

    for b in range(NBUF):
        wait_gather(NCHA - NBUF + b, b)
        start_scatter(NCHA - NBUF + b, b)
    for b in range(NBUF):
        wait_scatter(NCHA - NBUF + b, b)

    plsc.subcore_barrier()

    @pl.loop(0, 6)
    def _write_out(k):
        pltpu.sync_copy(agg_sh.at[pl.ds(s * DSEG + k * (DSEG // 6), DSEG // 6)],
                        stg)
        pltpu.sync_copy(stg,
                        out_hbm.at[pl.ds(s * DSEG + k * (DSEG // 6), DSEG // 6),
                                   pl.ds(c * D2, D2)])

    @pl.when(s == NS - 1)
    def _write_tail():
        pltpu.sync_copy(agg_sh.at[pl.ds(NS * DSEG, DTAIL)],
                        stg.at[pl.ds(0, DTAIL)])
        pltpu.sync_copy(stg.at[pl.ds(0, DTAIL)],
                        out_hbm.at[pl.ds(NS * DSEG, DTAIL),
                                   pl.ds(c * D2, D2)])


@functools.cache
def _sc_agg():
    return pl.kernel(
        _sc_agg_body,
        out_type=jax.ShapeDtypeStruct((N, D), jnp.float32),
        mesh=_mesh(),
        compiler_params=pltpu.CompilerParams(use_tc_tiling_on_sc=False),
        scratch_types=[
            pltpu.VMEM((NCHA, ACHUNK), jnp.int32),
            pltpu.VMEM((NCHA, ACHUNK), jnp.int32),
            pltpu.VMEM((ACHUNK, D2), jnp.float32),
            pltpu.VMEM((ACHUNK, D2), jnp.float32),
            pltpu.VMEM((ACHUNK, D2), jnp.float32),
            pltpu.VMEM((ACHUNK, D2), jnp.float32),
            pltpu.VMEM((DSEG // 6, D2), jnp.float32),
            pltpu.VMEM_SHARED((N, D2), jnp.float32),
        ] + [pltpu.SemaphoreType.DMA] * 8,
    )


def _tc1_body(x_ref, w1_ref, od_ref, id_ref, h_ref, ns_ref, nd_ref):
    dpo = od_ref[...]
    dpi = id_ref[...]
    od = dpo[0] + dpo[1]
    ind = dpi[0] + dpi[1]
    ns = jnp.where(od > 0, lax.rsqrt(jnp.maximum(od, 1e-12)), 0.0)
    nd = jnp.where(ind > 0, lax.rsqrt(jnp.maximum(ind, 1e-12)), 0.0)
    ns_ref[...] = ns
    nd_ref[...] = nd
    xs = x_ref[...] * ns[:, None]
    h_ref[...] = jnp.dot(xs, w1_ref[...], preferred_element_type=jnp.float32)


_tc1 = pl.pallas_call(
    _tc1_body,
    out_shape=[
        jax.ShapeDtypeStruct((N, D), jnp.float32),
        jax.ShapeDtypeStruct((N,), jnp.float32),
        jax.ShapeDtypeStruct((N,), jnp.float32),
    ],
)


def _tc2_body(agg_ref, ns_ref, nd_ref, b1_ref, w2_ref, out_ref):
    a = agg_ref[...]
    h1 = jnp.maximum(a * nd_ref[...][:, None] + b1_ref[...], 0.0)
    out_ref[...] = jnp.dot(h1 * ns_ref[...][:, None], w2_ref[...],
                           preferred_element_type=jnp.float32)


_tc2 = pl.pallas_call(
    _tc2_body,
    out_shape=jax.ShapeDtypeStruct((N, D), jnp.float32),
)


def _tc3_body(agg_ref, nd_ref, b2_ref, out_ref):
    out_ref[...] = agg_ref[...] * nd_ref[...][:, None] + b2_ref[...]


_tc3 = pl.pallas_call(
    _tc3_body,
    out_shape=jax.ShapeDtypeStruct((N, D), jnp.float32),
)


def kernel(x, edge_index, W1, b1, W2, b2):
    ei = edge_index.astype(jnp.int32)
    src_a = ei[0].reshape(NS, NCHA, ACHUNK)
    dst_a = ei[1].reshape(NS, NCHA, ACHUNK)
    b1r = b1.reshape(1, D)
    b2r = b2.reshape(1, D)

    sc_deg = _sc_deg()
    sc_agg = _sc_agg()
    od_part, id_part = sc_deg(src_a, dst_a)
    src2 = jnp.stack([src_a * 2, src_a * 2 + 1])
    h1p, ns, nd = _tc1(x, W1, od_part.reshape(NC, N), id_part.reshape(NC, N))
    agg1 = sc_agg(h1p.reshape(2 * N, D2), src2, dst_a)
    h2p = _tc2(agg1, ns, nd, b1r, W2)
    agg2 = sc_agg(h2p.reshape(2 * N, D2), src2, dst_a)
    return _tc3(agg2, nd, b2r)


# speedup vs baseline: 1.0257x; 1.0257x over previous
import functools

import jax
import jax.numpy as jnp
from jax import lax
from jax.experimental import pallas as pl
from jax.experimental.pallas import tpu as pltpu
from jax.experimental.pallas import tpu_sc as plsc

N = 10000
E = 320000
D = 128

NC = 2
NS = 16
NW = NC * NS
D2 = D // 2
ACHUNK = 125
NCHUNK = E // NW // ACHUNK
NCHA = E // NS // ACHUNK
NBUF = 4

DSEG = 624
DTAIL = N - NS * DSEG

@functools.cache
def _mesh():
    return plsc.VectorSubcoreMesh(
        core_axis_name="c", subcore_axis_name="s",
        num_cores=NC, num_subcores=NS)


def _zeros16():
    return jnp.zeros((16,), jnp.float32)


def _sc_deg_body(src_hbm, dst_hbm, od_hbm, id_hbm,
                 idxv, idxv2, onesv, zbuf, od_sh, id_sh, dsem, dsem2):
    c = lax.axis_index("c")
    s = lax.axis_index("s")
    w = c * NS + s

    @pl.loop(0, 40)
    def _zero_zbuf(i):
        zbuf[pl.ds(i * 16, 16)] = _zeros16()

    for j in range(128 // 16):
        onesv[pl.ds(j * 16, 16)] = jnp.ones((16,), jnp.float32)

    pltpu.sync_copy(zbuf.at[pl.ds(0, DSEG)], od_sh.at[pl.ds(s * DSEG, DSEG)])
    pltpu.sync_copy(zbuf.at[pl.ds(0, DSEG)], id_sh.at[pl.ds(s * DSEG, DSEG)])

    @pl.when(s == NS - 1)
    def _zero_tail():
        pltpu.sync_copy(zbuf.at[pl.ds(0, DTAIL)],
                        od_sh.at[pl.ds(NS * DSEG, DTAIL)])
        pltpu.sync_copy(zbuf.at[pl.ds(0, DTAIL)],
                        id_sh.at[pl.ds(NS * DSEG, DTAIL)])

    plsc.subcore_barrier()

    pltpu.sync_copy(src_hbm.at[s, pl.ds(c * NCHUNK, NCHUNK)], idxv)
    pltpu.sync_copy(dst_hbm.at[s, pl.ds(c * NCHUNK, NCHUNK)], idxv2)

    ones = onesv.at[pl.ds(0, ACHUNK)]

    @pl.loop(0, NCHUNK, step=8)
    def _scatter_deg(j):
        for b in range(8):
            pltpu.async_copy(ones, od_sh.at[idxv.at[j + b]], dsem, add=True)
            pltpu.async_copy(ones, id_sh.at[idxv2.at[j + b]], dsem2, add=True)
        for b in range(8):
            pltpu.make_async_copy(ones, od_sh.at[idxv.at[j + b]], dsem).wait()
            pltpu.make_async_copy(ones, id_sh.at[idxv2.at[j + b]],
                                  dsem2).wait()

    plsc.subcore_barrier()

    pltpu.sync_copy(od_sh.at[pl.ds(s * DSEG, DSEG)], zbuf.at[pl.ds(0, DSEG)])
    pltpu.sync_copy(zbuf.at[pl.ds(0, DSEG)],
                    od_hbm.at[pl.ds(c * N + s * DSEG, DSEG)])
    pltpu.sync_copy(id_sh.at[pl.ds(s * DSEG, DSEG)], zbuf.at[pl.ds(0, DSEG)])
    pltpu.sync_copy(zbuf.at[pl.ds(0, DSEG)],
                    id_hbm.at[pl.ds(c * N + s * DSEG, DSEG)])

    @pl.when(s == NS - 1)
    def _write_tail():
        pltpu.sync_copy(od_sh.at[pl.ds(NS * DSEG, DTAIL)],
                        zbuf.at[pl.ds(0, DTAIL)])
        pltpu.sync_copy(zbuf.at[pl.ds(0, DTAIL)],
                        od_hbm.at[pl.ds(c * N + NS * DSEG, DTAIL)])
        pltpu.sync_copy(id_sh.at[pl.ds(NS * DSEG, DTAIL)],
                        zbuf.at[pl.ds(0, DTAIL)])
        pltpu.sync_copy(zbuf.at[pl.ds(0, DTAIL)],
                        id_hbm.at[pl.ds(c * N + NS * DSEG, DTAIL)])


@functools.cache
def _sc_deg():
    return pl.kernel(
        _sc_deg_body,
        out_type=[
            jax.ShapeDtypeStruct((NC * N,), jnp.float32),
            jax.ShapeDtypeStruct((NC * N,), jnp.float32),
        ],
        mesh=_mesh(),
        scratch_types=[
            pltpu.VMEM((NCHUNK, ACHUNK), jnp.int32),
            pltpu.VMEM((NCHUNK, ACHUNK), jnp.int32),
            pltpu.VMEM((128,), jnp.float32),
            pltpu.VMEM((640,), jnp.float32),
            pltpu.VMEM_SHARED((N,), jnp.float32),
            pltpu.VMEM_SHARED((N,), jnp.float32),
            pltpu.SemaphoreType.DMA,
            pltpu.SemaphoreType.DMA,
        ],
    )


def _sc_agg_body(h_hbm, src2_hbm, dst_hbm, out_hbm,
                 srcv, dstv, buf0, buf1, buf2, buf3,
                 stg, stg2, agg_sh, gsem0, gsem1, gsem2, gsem3,
                 ssem0, ssem1, ssem2, ssem3, wsem):
    c = lax.axis_index("c")
    s = lax.axis_index("s")
    bufs = (buf0, buf1, buf2, buf3)
    gsems = (gsem0, gsem1, gsem2, gsem3)
    ssems = (ssem0, ssem1, ssem2, ssem3)

    pltpu.async_copy(src2_hbm.at[c, s], srcv, gsem0)
    pltpu.async_copy(dst_hbm.at[s], dstv, gsem1)

    @pl.loop(0, DSEG // 6)
    def _zero_stg(r):
        for j in range(D2 // 16):
            stg[r, pl.ds(j * 16, 16)] = _zeros16()

    @pl.loop(0, 6)
    def _zero_agg(k):
        pltpu.sync_copy(stg,
                        agg_sh.at[pl.ds(s * DSEG + k * (DSEG // 6), DSEG // 6)])

    @pl.when(s == NS - 1)
    def _zero_tail():
        pltpu.sync_copy(stg.at[pl.ds(0, DTAIL)],
                        agg_sh.at[pl.ds(NS * DSEG, DTAIL)])

    pltpu.make_async_copy(src2_hbm.at[c, s], srcv, gsem0).wait()
    pltpu.make_async_copy(dst_hbm.at[s], dstv, gsem1).wait()

    plsc.subcore_barrier()

    hc = h_hbm

    def start_gather(j, b):
        pltpu.async_copy(hc.at[srcv.at[j]], bufs[b], gsems[b])

    def wait_gather(j, b):
        pltpu.make_async_copy(hc.at[srcv.at[j]], bufs[b], gsems[b]).wait()

    def start_scatter(j, b):
        pltpu.async_copy(bufs[b], agg_sh.at[dstv.at[j]], ssems[b], add=True)

    def wait_scatter(j, b):
        pltpu.make_async_copy(bufs[b], agg_sh.at[dstv.at[j]],
                              ssems[b]).wait()

    for b in range(NBUF):
        start_gather(b, b)

    @pl.loop(0, NCHA - NBUF, step=NBUF)
    def _edge_loop(j):
        for b in range(NBUF):
            wait_gather(j + b, b)
            start_scatter(j + b, b)
        for b in range(NBUF):
            wait_scatter(j + b, b)
            start_gather(j + NBUF + b, b)

    for b in range(NBUF):
        wait_gather(NCHA - NBUF + b, b)
        start_scatter(NCHA - NBUF + b, b)
    for b in range(NBUF):
        wait_scatter(NCHA - NBUF + b, b)

    plsc.subcore_barrier()

    stgs = (stg, stg2)

    @pl.loop(0, 6, step=2)
    def _write_out(k):
        for b in range(2):
            r0 = s * DSEG + (k + b) * (DSEG // 6)
            pltpu.sync_copy(agg_sh.at[pl.ds(r0, DSEG // 6)], stgs[b])
            pltpu.async_copy(stgs[b],
                             out_hbm.at[pl.ds(r0, DSEG // 6),
                                        pl.ds(c * D2, D2)], wsem)
        for b in range(2):
            r0 = s * DSEG + (k + b) * (DSEG // 6)
            pltpu.make_async_copy(stgs[b],
                                  out_hbm.at[pl.ds(r0, DSEG // 6),
                                             pl.ds(c * D2, D2)], wsem).wait()

    @pl.when(s == NS - 1)
    def _write_tail():
        pltpu.sync_copy(agg_sh.at[pl.ds(NS * DSEG, DTAIL)],
                        stg.at[pl.ds(0, DTAIL)])
        pltpu.sync_copy(stg.at[pl.ds(0, DTAIL)],
                        out_hbm.at[pl.ds(NS * DSEG, DTAIL),
                                   pl.ds(c * D2, D2)])


@functools.cache
def _sc_agg():
    return pl.kernel(
        _sc_agg_body,
        out_type=jax.ShapeDtypeStruct((N, D), jnp.float32),
        mesh=_mesh(),
        compiler_params=pltpu.CompilerParams(use_tc_tiling_on_sc=False),
        scratch_types=[
            pltpu.VMEM((NCHA, ACHUNK), jnp.int32),
            pltpu.VMEM((NCHA, ACHUNK), jnp.int32),
            pltpu.VMEM((ACHUNK, D2), jnp.float32),
            pltpu.VMEM((ACHUNK, D2), jnp.float32),
            pltpu.VMEM((ACHUNK, D2), jnp.float32),
            pltpu.VMEM((ACHUNK, D2), jnp.float32),
            pltpu.VMEM((DSEG // 6, D2), jnp.float32),
            pltpu.VMEM((DSEG // 6, D2), jnp.float32),
            pltpu.VMEM_SHARED((N, D2), jnp.float32),
        ] + [pltpu.SemaphoreType.DMA] * 9,
    )


def _tc1_body(x_ref, w1_ref, od_ref, id_ref, h_ref, ns_ref, nd_ref):
    dpo = od_ref[...]
    dpi = id_ref[...]
    od = dpo[0] + dpo[1]
    ind = dpi[0] + dpi[1]
    ns = jnp.where(od > 0, lax.rsqrt(jnp.maximum(od, 1e-12)), 0.0)
    nd = jnp.where(ind > 0, lax.rsqrt(jnp.maximum(ind, 1e-12)), 0.0)
    ns_ref[...] = ns
    nd_ref[...] = nd
    xs = x_ref[...] * ns[:, None]
    h_ref[...] = jnp.dot(xs, w1_ref[...], preferred_element_type=jnp.float32)


_tc1 = pl.pallas_call(
    _tc1_body,
    out_shape=[
        jax.ShapeDtypeStruct((N, D), jnp.float32),
        jax.ShapeDtypeStruct((N,), jnp.float32),
        jax.ShapeDtypeStruct((N,), jnp.float32),
    ],
)


def _tc2_body(agg_ref, ns_ref, nd_ref, b1_ref, w2_ref, out_ref):
    a = agg_ref[...]
    h1 = jnp.maximum(a * nd_ref[...][:, None] + b1_ref[...], 0.0)
    out_ref[...] = jnp.dot(h1 * ns_ref[...][:, None], w2_ref[...],
                           preferred_element_type=jnp.float32)


_tc2 = pl.pallas_call(
    _tc2_body,
    out_shape=jax.ShapeDtypeStruct((N, D), jnp.float32),
)


def _tc3_body(agg_ref, nd_ref, b2_ref, out_ref):
    out_ref[...] = agg_ref[...] * nd_ref[...][:, None] + b2_ref[...]


_tc3 = pl.pallas_call(
    _tc3_body,
    out_shape=jax.ShapeDtypeStruct((N, D), jnp.float32),
)


def kernel(x, edge_index, W1, b1, W2, b2):
    ei = edge_index.astype(jnp.int32)
    src_a = ei[0].reshape(NS, NCHA, ACHUNK)
    dst_a = ei[1].reshape(NS, NCHA, ACHUNK)
    b1r = b1.reshape(1, D)
    b2r = b2.reshape(1, D)

    sc_deg = _sc_deg()
    sc_agg = _sc_agg()
    od_part, id_part = sc_deg(src_a, dst_a)
    src2 = jnp.stack([src_a * 2, src_a * 2 + 1])
    h1p, ns, nd = _tc1(x, W1, od_part.reshape(NC, N), id_part.reshape(NC, N))
    agg1 = sc_agg(h1p.reshape(2 * N, D2), src2, dst_a)
    h2p = _tc2(agg1, ns, nd, b1r, W2)
    agg2 = sc_agg(h2p.reshape(2 * N, D2), src2, dst_a)
    return _tc3(agg2, nd, b2r)
